# baseline (speedup 1.0000x reference)
import jax
import jax.numpy as jnp
from jax.experimental import pallas as pl

_B = 64
_C = 3
_HW = 512
_G = 51
_P = _G * _G
_K = _C * 100
_HID = 16
_NOBJ = 5
_D1 = 256
_D2 = 128
_EPS = 1e-5


def _conv_stage(p_ref, w1_ref, b1_ref, h_ref, ssum_ref, ssq_ref):
    b = pl.program_id(0)
    yt = jax.lax.dot_general(
        w1_ref[...], p_ref[0], (((0,), (1,)), ((), ())),
        preferred_element_type=jnp.float32)
    yt = yt + b1_ref[...]
    h_ref[...] = yt[None]

    @pl.when(b == 0)
    def _():
        ssum_ref[...] = jnp.zeros_like(ssum_ref)
        ssq_ref[...] = jnp.zeros_like(ssq_ref)

    ssum_ref[...] += jnp.sum(yt, axis=1, keepdims=True)
    ssq_ref[...] += jnp.sum(yt * yt, axis=1, keepdims=True)


def _mlp_stage(h_ref, scale_ref, shift_ref, w2_ref, b2_ref,
               fc1w_ref, fc1b_ref, fc2w_ref, fc2b_ref, lng_ref, lnb_ref,
               fc3w_ref, fc3b_ref, out_ref):
    h = h_ref[...]
    a = jnp.maximum(h * scale_ref[...][None] + shift_ref[...][None], 0.0)
    gt = jax.lax.dot_general(
        w2_ref[...], a, (((1,), (1,)), ((), ())),
        preferred_element_type=jnp.float32)
    gt = jax.nn.sigmoid(gt + b2_ref[...])
    g2 = gt.reshape(_NOBJ * _B, _P)
    z1 = jax.lax.dot_general(
        g2, fc1w_ref[...], (((1,), (0,)), ((), ())),
        preferred_element_type=jnp.float32)
    z1 = jnp.maximum(z1 + fc1b_ref[...], 0.0)
    z2 = jax.lax.dot_general(
        z1, fc2w_ref[...], (((1,), (0,)), ((), ())),
        preferred_element_type=jnp.float32)
    z2 = z2 + fc2b_ref[...]
    mu = jnp.mean(z2, axis=1, keepdims=True)
    var = jnp.mean(z2 * z2, axis=1, keepdims=True) - mu * mu
    z2 = (z2 - mu) * jax.lax.rsqrt(var + _EPS) * lng_ref[...] + lnb_ref[...]
    z2 = jnp.maximum(z2, 0.0)
    z3 = jax.lax.dot_general(
        z2, fc3w_ref[...], (((1,), (0,)), ((), ())),
        preferred_element_type=jnp.float32)
    out_ref[...] = z3 + fc3b_ref[...]


def kernel(obs, cnn1_w, cnn1_b, bn_g, bn_b, cnn2_w, cnn2_b,
           fc1_w, fc1_b, fc2_w, fc2_b, ln_g, ln_b, fc3_w, fc3_b):
    f32 = jnp.float32
    bf16 = jnp.bfloat16
    patches = (obs[:, :, :510, :510]
               .reshape(_B, _C, _G, 10, _G, 10)
               .transpose(0, 2, 4, 1, 3, 5)
               .reshape(_B, _P, _K)
               .astype(bf16))
    w1r = cnn1_w.transpose(1, 2, 3, 0).reshape(_K, _HID).astype(bf16)
    b1r = cnn1_b.reshape(_HID, 1)

    h, ssum, ssq = pl.pallas_call(
        _conv_stage,
        grid=(_B,),
        in_specs=[
            pl.BlockSpec((1, _P, _K), lambda b: (b, 0, 0)),
            pl.BlockSpec((_K, _HID), lambda b: (0, 0)),
            pl.BlockSpec((_HID, 1), lambda b: (0, 0)),
        ],
        out_specs=[
            pl.BlockSpec((1, _HID, _P), lambda b: (b, 0, 0)),
            pl.BlockSpec((_HID, 1), lambda b: (0, 0)),
            pl.BlockSpec((_HID, 1), lambda b: (0, 0)),
        ],
        out_shape=[
            jax.ShapeDtypeStruct((_B, _HID, _P), f32),
            jax.ShapeDtypeStruct((_HID, 1), f32),
            jax.ShapeDtypeStruct((_HID, 1), f32),
        ],
    )(patches, w1r, b1r)

    n = float(_B * _P)
    mean = ssum / n
    var = ssq / n - mean * mean
    scale = bn_g.reshape(_HID, 1) / jnp.sqrt(var + _EPS)
    shift = bn_b.reshape(_HID, 1) - mean * scale

    z3 = pl.pallas_call(
        _mlp_stage,
        out_shape=jax.ShapeDtypeStruct((_NOBJ * _B, _D2), f32),
    )(h, scale, shift,
      cnn2_w.reshape(_NOBJ, _HID), cnn2_b.reshape(_NOBJ, 1, 1),
      fc1_w.T, fc1_b.reshape(1, _D1),
      fc2_w.T, fc2_b.reshape(1, _D1),
      ln_g.reshape(1, _D1), ln_b.reshape(1, _D1),
      fc3_w.T, fc3_b.reshape(1, _D2))

    return z3.reshape(_NOBJ, _B, _D2).transpose(1, 0, 2)


# speedup vs baseline: 28.0085x; 28.0085x over previous
import jax
import jax.numpy as jnp
from jax.experimental import pallas as pl

_B = 64
_C = 3
_HW = 512
_G = 51
_P = _G * _G
_K = _C * 100
_HID = 16
_NOBJ = 5
_D1 = 256
_D2 = 128
_EPS = 1e-5


def _conv_stage(p_ref, w1_ref, b1_ref, h_ref, ssum_ref, ssq_ref):
    b = pl.program_id(0)
    x = p_ref[0][:, :510, :510].astype(jnp.bfloat16)
    x5 = x.reshape(_C, _G, 10, _G, 10)
    pt = x5.transpose(0, 2, 4, 1, 3).reshape(_K, _P)
    yt = jax.lax.dot_general(
        w1_ref[...], pt, (((1,), (0,)), ((), ())),
        preferred_element_type=jnp.float32)
    yt = yt + b1_ref[...]
    h_ref[...] = yt[None]

    @pl.when(b == 0)
    def _():
        ssum_ref[...] = jnp.zeros_like(ssum_ref)
        ssq_ref[...] = jnp.zeros_like(ssq_ref)

    ssum_ref[...] += jnp.sum(yt, axis=1, keepdims=True)
    ssq_ref[...] += jnp.sum(yt * yt, axis=1, keepdims=True)


def _mlp_stage(h_ref, scale_ref, shift_ref, w2_ref, b2_ref,
               fc1w_ref, fc1b_ref, fc2w_ref, fc2b_ref, lng_ref, lnb_ref,
               fc3w_ref, fc3b_ref, out_ref):
    h = h_ref[...]
    a = jnp.maximum(h * scale_ref[...][None] + shift_ref[...][None], 0.0)
    gt = jax.lax.dot_general(
        w2_ref[...], a, (((1,), (1,)), ((), ())),
        preferred_element_type=jnp.float32)
    gt = jax.nn.sigmoid(gt + b2_ref[...])
    g2 = gt.reshape(_NOBJ * _B, _P)
    z1 = jax.lax.dot_general(
        g2, fc1w_ref[...], (((1,), (0,)), ((), ())),
        preferred_element_type=jnp.float32)
    z1 = jnp.maximum(z1 + fc1b_ref[...], 0.0)
    z2 = jax.lax.dot_general(
        z1, fc2w_ref[...], (((1,), (0,)), ((), ())),
        preferred_element_type=jnp.float32)
    z2 = z2 + fc2b_ref[...]
    mu = jnp.mean(z2, axis=1, keepdims=True)
    var = jnp.mean(z2 * z2, axis=1, keepdims=True) - mu * mu
    z2 = (z2 - mu) * jax.lax.rsqrt(var + _EPS) * lng_ref[...] + lnb_ref[...]
    z2 = jnp.maximum(z2, 0.0)
    z3 = jax.lax.dot_general(
        z2, fc3w_ref[...], (((1,), (0,)), ((), ())),
        preferred_element_type=jnp.float32)
    out_ref[...] = z3 + fc3b_ref[...]


def kernel(obs, cnn1_w, cnn1_b, bn_g, bn_b, cnn2_w, cnn2_b,
           fc1_w, fc1_b, fc2_w, fc2_b, ln_g, ln_b, fc3_w, fc3_b):
    f32 = jnp.float32
    bf16 = jnp.bfloat16
    w1r = cnn1_w.reshape(_HID, _K).astype(bf16)
    b1r = cnn1_b.reshape(_HID, 1)

    h, ssum, ssq = pl.pallas_call(
        _conv_stage,
        grid=(_B,),
        in_specs=[
            pl.BlockSpec((1, _C, _HW, _HW), lambda b: (b, 0, 0, 0)),
            pl.BlockSpec((_HID, _K), lambda b: (0, 0)),
            pl.BlockSpec((_HID, 1), lambda b: (0, 0)),
        ],
        out_specs=[
            pl.BlockSpec((1, _HID, _P), lambda b: (b, 0, 0)),
            pl.BlockSpec((_HID, 1), lambda b: (0, 0)),
            pl.BlockSpec((_HID, 1), lambda b: (0, 0)),
        ],
        out_shape=[
            jax.ShapeDtypeStruct((_B, _HID, _P), f32),
            jax.ShapeDtypeStruct((_HID, 1), f32),
            jax.ShapeDtypeStruct((_HID, 1), f32),
        ],
    )(obs, w1r, b1r)

    n = float(_B * _P)
    mean = ssum / n
    var = ssq / n - mean * mean
    scale = bn_g.reshape(_HID, 1) / jnp.sqrt(var + _EPS)
    shift = bn_b.reshape(_HID, 1) - mean * scale

    z3 = pl.pallas_call(
        _mlp_stage,
        out_shape=jax.ShapeDtypeStruct((_NOBJ * _B, _D2), f32),
    )(h, scale, shift,
      cnn2_w.reshape(_NOBJ, _HID), cnn2_b.reshape(_NOBJ, 1, 1),
      fc1_w.T, fc1_b.reshape(1, _D1),
      fc2_w.T, fc2_b.reshape(1, _D1),
      ln_g.reshape(1, _D1), ln_b.reshape(1, _D1),
      fc3_w.T, fc3_b.reshape(1, _D2))

    return z3.reshape(_NOBJ, _B, _D2).transpose(1, 0, 2)
